# baseline (speedup 1.0000x reference)
import functools

import jax
import jax.numpy as jnp
from jax import lax
from jax.experimental import pallas as pl
from jax.experimental.pallas import tpu as pltpu
from jax.experimental.pallas import tpu_sc as plsc

NC, NS = 2, 16
NW = NC * NS
BATCH, SEQ = 4096, 200
POS_DIM = 64
VOCAB = 50
B = BATCH * SEQ
BPW = B // NW
L = 16

_mesh = plsc.VectorSubcoreMesh(
    core_axis_name="c", subcore_axis_name="s", num_cores=NC, num_subcores=NS
)


@functools.partial(
    pl.kernel,
    out_type=jax.ShapeDtypeStruct((SEQ, 8, NW, 1024), jnp.float32),
    mesh=_mesh,
    scratch_types=[
        pltpu.VMEM((BPW,), jnp.int32),
        pltpu.VMEM((VOCAB * POS_DIM,), jnp.float32),
        pltpu.VMEM((POS_DIM * 64,), jnp.float32),
        pltpu.VMEM((128,), jnp.int32),
        [pltpu.VMEM((8 * 1024,), jnp.float32) for _ in range(2)],
        [pltpu.SemaphoreType.DMA for _ in range(2)],
    ],
    compiler_params=pltpu.CompilerParams(use_tc_tiling_on_sc=False,
                                         needs_layout_passes=False),
)
def _pos2vec(idx_hbm, tab_hbm, out_hbm, idx_v, tab_v, tabx_v, idxbuf,
             outv, sems):
    wid = lax.axis_index("s") * NC + lax.axis_index("c")

    pltpu.sync_copy(idx_hbm.at[pl.ds(wid * BPW, BPW)], idx_v)
    pltpu.sync_copy(tab_hbm, tab_v)

    lanes = jnp.arange(L, dtype=jnp.int32)
    for vv in range(4):
        vcl = jnp.minimum(lanes + vv * L, VOCAB - 1) * POS_DIM
        for k in range(POS_DIM):
            tabx_v[pl.ds(k * 64 + vv * L, L)] = plsc.load_gather(tab_v,
                                                                 [vcl + k])

    sbase = [(lanes + bbv * L) * SEQ for bbv in range(8)]

    def emit(s, p):
        sv = jnp.zeros((L,), jnp.int32) + s
        for bbv in range(8):
            idxbuf[pl.ds(bbv * L, L)] = plsc.load_gather(idx_v,
                                                         [sbase[bbv] + sv])

        @plsc.parallel_loop(0, 512, unroll=8)
        def _(j):
            k = j & 63
            iv = idxbuf[pl.ds((j >> 6) * L, L)]
            g = plsc.load_gather(tabx_v, [iv + k * POS_DIM])
            outv[p][pl.ds((k >> 3) * 1024 + (k & 7) * 128 + (j >> 6) * L,
                          L)] = g

        for co in range(8):
            pltpu.async_copy(outv[p].at[pl.ds(co * 1024, 1024)],
                             out_hbm.at[s, co, wid], sems[p])

    def drain(p):
        for co in range(8):
            pltpu.make_async_copy(outv[p].at[pl.ds(co * 1024, 1024)],
                                  out_hbm.at[0, co, wid], sems[p]).wait()

    def body(s2, carry):
        for p in range(2):
            @pl.when(s2 >= 1)
            def _():
                drain(p)
            emit(2 * s2 + p, p)
        return carry

    lax.fori_loop(0, SEQ // 2, body, 0, unroll=False)
    drain(0)
    drain(1)


def kernel(indices, table):
    flat = indices.reshape(-1).astype(jnp.int32)
    t5 = _pos2vec(flat, table.reshape(-1))
    return (t5.reshape(SEQ, 8, NW, 8, 128)
              .transpose(2, 4, 0, 1, 3)
              .reshape(BATCH, SEQ, POS_DIM))


# speedup vs baseline: 1.2678x; 1.2678x over previous
import functools

import jax
import jax.numpy as jnp
from jax import lax
from jax.experimental import pallas as pl
from jax.experimental.pallas import tpu as pltpu
from jax.experimental.pallas import tpu_sc as plsc

NC, NS = 2, 16
NW = NC * NS
BATCH, SEQ = 4096, 200
POS_DIM = 64
VOCAB = 50
B = BATCH * SEQ
BPW = B // NW
L = 16

_mesh = plsc.VectorSubcoreMesh(
    core_axis_name="c", subcore_axis_name="s", num_cores=NC, num_subcores=NS
)


@functools.partial(
    pl.kernel,
    out_type=jax.ShapeDtypeStruct((SEQ, 8, NW, 1024), jnp.float32),
    mesh=_mesh,
    scratch_types=[
        pltpu.VMEM((BPW,), jnp.int32),
        pltpu.VMEM((VOCAB * POS_DIM,), jnp.float32),
        pltpu.VMEM((POS_DIM * 64,), jnp.float32),
        pltpu.VMEM((128,), jnp.int32),
        [pltpu.VMEM((8 * 1024,), jnp.float32) for _ in range(2)],
        [pltpu.SemaphoreType.DMA for _ in range(2)],
    ],
    compiler_params=pltpu.CompilerParams(use_tc_tiling_on_sc=False,
                                         needs_layout_passes=False),
)
def _pos2vec(idx_hbm, tab_hbm, out_hbm, idx_v, tab_v, tabx_v, idxbuf,
             outv, sems):
    wid = lax.axis_index("s") * NC + lax.axis_index("c")

    pltpu.sync_copy(idx_hbm.at[pl.ds(wid * BPW, BPW)], idx_v)
    pltpu.sync_copy(tab_hbm, tab_v)

    lanes = jnp.arange(L, dtype=jnp.int32)
    for vv in range(4):
        vcl = jnp.minimum(lanes + vv * L, VOCAB - 1) * POS_DIM
        for k in range(POS_DIM):
            tabx_v[pl.ds(k * 64 + vv * L, L)] = plsc.load_gather(tab_v,
                                                                 [vcl + k])

    sbase = [(lanes + bbv * L) * SEQ for bbv in range(8)]

    def emit(s, p):
        sv = jnp.zeros((L,), jnp.int32) + s
        for bbv in range(8):
            idxbuf[pl.ds(bbv * L, L)] = plsc.load_gather(idx_v,
                                                         [sbase[bbv] + sv])

        @plsc.parallel_loop(0, 512, unroll=16)
        def _(j):
            iv = idxbuf[pl.ds((j & 7) * L, L)]
            g = plsc.load_gather(tabx_v, [iv + (j >> 3) * POS_DIM])
            outv[p][pl.ds(j * L, L)] = g

        for co in range(8):
            pltpu.async_copy(outv[p].at[pl.ds(co * 1024, 1024)],
                             out_hbm.at[s, co, wid], sems[p])

    def drain(p):
        for co in range(8):
            pltpu.make_async_copy(outv[p].at[pl.ds(co * 1024, 1024)],
                                  out_hbm.at[0, co, wid], sems[p]).wait()

    def body(s2, carry):
        for p in range(2):
            @pl.when(s2 >= 1)
            def _():
                drain(p)
            emit(2 * s2 + p, p)
        return carry

    lax.fori_loop(0, SEQ // 2, body, 0, unroll=False)
    drain(0)
    drain(1)


def kernel(indices, table):
    flat = indices.reshape(-1).astype(jnp.int32)
    t5 = _pos2vec(flat, table.reshape(-1))
    return (t5.reshape(SEQ, 8, NW, 8, 128)
              .transpose(2, 4, 0, 1, 3)
              .reshape(BATCH, SEQ, POS_DIM))


# speedup vs baseline: 1.3632x; 1.0752x over previous
import functools

import jax
import jax.numpy as jnp
from jax import lax
from jax.experimental import pallas as pl
from jax.experimental.pallas import tpu as pltpu
from jax.experimental.pallas import tpu_sc as plsc

NC, NS = 2, 16
NW = NC * NS
BATCH, SEQ = 4096, 200
POS_DIM = 64
VOCAB = 50
B = BATCH * SEQ
BPW = B // NW
L = 16

_mesh = plsc.VectorSubcoreMesh(
    core_axis_name="c", subcore_axis_name="s", num_cores=NC, num_subcores=NS
)


@functools.partial(
    pl.kernel,
    out_type=jax.ShapeDtypeStruct((SEQ, 8, NW, 1024), jnp.float32),
    mesh=_mesh,
    scratch_types=[
        pltpu.VMEM((BPW,), jnp.int32),
        pltpu.VMEM((VOCAB * POS_DIM,), jnp.float32),
        pltpu.VMEM((POS_DIM * 64,), jnp.float32),
        pltpu.VMEM((128,), jnp.int32),
        [pltpu.VMEM((8 * 1024,), jnp.float32) for _ in range(2)],
        [pltpu.SemaphoreType.DMA for _ in range(2)],
    ],
    compiler_params=pltpu.CompilerParams(use_tc_tiling_on_sc=False,
                                         needs_layout_passes=False),
)
def _pos2vec(idx_hbm, tab_hbm, out_hbm, idx_v, tab_v, tabx_v, idxbuf,
             outv, sems):
    wid = lax.axis_index("s") * NC + lax.axis_index("c")

    pltpu.sync_copy(idx_hbm.at[pl.ds(wid * BPW, BPW)], idx_v)
    pltpu.sync_copy(tab_hbm, tab_v)

    lanes = jnp.arange(L, dtype=jnp.int32)
    for vv in range(4):
        vcl = jnp.minimum(lanes + vv * L, VOCAB - 1) * POS_DIM
        for k in range(POS_DIM):
            tabx_v[pl.ds(k * 64 + vv * L, L)] = plsc.load_gather(tab_v,
                                                                 [vcl + k])

    sbase = [(lanes + bbv * L) * SEQ for bbv in range(8)]

    def emit(s, p):
        sv = jnp.zeros((L,), jnp.int32) + s
        for bbv in range(8):
            idxbuf[pl.ds(bbv * L, L)] = plsc.load_gather(idx_v,
                                                         [sbase[bbv] + sv])

        @plsc.parallel_loop(0, 512, unroll=32)
        def _(j):
            iv = idxbuf[pl.ds((j & 7) * L, L)]
            g = plsc.load_gather(tabx_v, [iv + (j >> 3) * POS_DIM])
            outv[p][pl.ds(j * L, L)] = g

        for co in range(8):
            pltpu.async_copy(outv[p].at[pl.ds(co * 1024, 1024)],
                             out_hbm.at[s, co, wid], sems[p])

    def drain(p):
        for co in range(8):
            pltpu.make_async_copy(outv[p].at[pl.ds(co * 1024, 1024)],
                                  out_hbm.at[0, co, wid], sems[p]).wait()

    def body(s2, carry):
        for p in range(2):
            @pl.when(s2 >= 1)
            def _():
                drain(p)
            emit(2 * s2 + p, p)
        return carry

    lax.fori_loop(0, SEQ // 2, body, 0, unroll=False)
    drain(0)
    drain(1)


def kernel(indices, table):
    flat = indices.reshape(-1).astype(jnp.int32)
    t5 = _pos2vec(flat, table.reshape(-1))
    return (t5.reshape(SEQ, 8, NW, 8, 128)
              .transpose(2, 4, 0, 1, 3)
              .reshape(BATCH, SEQ, POS_DIM))


# speedup vs baseline: 1.3885x; 1.0185x over previous
import functools

import jax
import jax.numpy as jnp
from jax import lax
from jax.experimental import pallas as pl
from jax.experimental.pallas import tpu as pltpu
from jax.experimental.pallas import tpu_sc as plsc

NC, NS = 2, 16
NW = NC * NS
BATCH, SEQ = 4096, 200
POS_DIM = 64
VOCAB = 50
B = BATCH * SEQ
BPW = B // NW
L = 16

_mesh = plsc.VectorSubcoreMesh(
    core_axis_name="c", subcore_axis_name="s", num_cores=NC, num_subcores=NS
)


@functools.partial(
    pl.kernel,
    out_type=jax.ShapeDtypeStruct((SEQ, 8, NW, 1024), jnp.float32),
    mesh=_mesh,
    scratch_types=[
        pltpu.VMEM((BPW,), jnp.int32),
        pltpu.VMEM((VOCAB * POS_DIM,), jnp.float32),
        pltpu.VMEM((POS_DIM * 64,), jnp.float32),
        pltpu.VMEM((128,), jnp.int32),
        [pltpu.VMEM((8 * 1024,), jnp.float32) for _ in range(2)],
        [pltpu.SemaphoreType.DMA for _ in range(2)],
    ],
    compiler_params=pltpu.CompilerParams(use_tc_tiling_on_sc=False,
                                         needs_layout_passes=False),
)
def _pos2vec(idx_hbm, tab_hbm, out_hbm, idx_v, tab_v, tabx_v, idxbuf,
             outv, sems):
    wid = lax.axis_index("s") * NC + lax.axis_index("c")

    pltpu.sync_copy(idx_hbm.at[pl.ds(wid * BPW, BPW)], idx_v)
    pltpu.sync_copy(tab_hbm, tab_v)

    lanes = jnp.arange(L, dtype=jnp.int32)
    for vv in range(4):
        vcl = jnp.minimum(lanes + vv * L, VOCAB - 1) * POS_DIM
        for k in range(POS_DIM):
            tabx_v[pl.ds(k * 64 + vv * L, L)] = plsc.load_gather(tab_v,
                                                                 [vcl + k])

    sbase = [(lanes + bbv * L) * SEQ for bbv in range(8)]

    def emit(s, p):
        sv = jnp.zeros((L,), jnp.int32) + s
        for bbv in range(8):
            idxbuf[pl.ds(bbv * L, L)] = plsc.load_gather(idx_v,
                                                         [sbase[bbv] + sv])

        @plsc.parallel_loop(0, 512, unroll=64)
        def _(j):
            iv = idxbuf[pl.ds((j & 7) * L, L)]
            g = plsc.load_gather(tabx_v, [iv + (j >> 3) * POS_DIM])
            outv[p][pl.ds(j * L, L)] = g

        for co in range(8):
            pltpu.async_copy(outv[p].at[pl.ds(co * 1024, 1024)],
                             out_hbm.at[s, co, wid], sems[p])

    def drain(p):
        for co in range(8):
            pltpu.make_async_copy(outv[p].at[pl.ds(co * 1024, 1024)],
                                  out_hbm.at[0, co, wid], sems[p]).wait()

    def body(s2, carry):
        for p in range(2):
            @pl.when(s2 >= 1)
            def _():
                drain(p)
            emit(2 * s2 + p, p)
        return carry

    lax.fori_loop(0, SEQ // 2, body, 0, unroll=False)
    drain(0)
    drain(1)


def kernel(indices, table):
    flat = indices.reshape(-1).astype(jnp.int32)
    t5 = _pos2vec(flat, table.reshape(-1))
    return (t5.reshape(SEQ, 8, NW, 8, 128)
              .transpose(2, 4, 0, 1, 3)
              .reshape(BATCH, SEQ, POS_DIM))
